# baseline (speedup 1.0000x reference)
import functools

import jax
import jax.numpy as jnp
from jax import lax
from jax.experimental import pallas as pl
from jax.experimental.pallas import tpu as pltpu
from jax.experimental.pallas import tpu_sc as plsc

_L = 128
_R = 8


def _make_sc_tile_gather(n_tiles: int, ni: int, d: int):
    info = plsc.get_sparse_core_info()
    nw = info.num_cores * info.num_subcores
    assert n_tiles % nw == 0
    tpw = n_tiles // nw

    mesh = plsc.VectorSubcoreMesh(core_axis_name="c", subcore_axis_name="s")

    @functools.partial(
        pl.kernel,
        mesh=mesh,
        compiler_params=pltpu.CompilerParams(use_tc_tiling_on_sc=False),
        out_type=jax.ShapeDtypeStruct((n_tiles, ni, d), jnp.float32),
        scratch_types=[
            pltpu.VMEM((tpw, ni), jnp.int32),
            pltpu.VMEM((ni, d), jnp.float32),
            pltpu.VMEM((ni, d), jnp.float32),
            pltpu.SemaphoreType.DMA,
            pltpu.SemaphoreType.DMA,
            pltpu.SemaphoreType.DMA,
            pltpu.SemaphoreType.DMA,
        ],
    )
    def sc_gather(idx_hbm, tab_hbm, out_hbm, idx_v, buf_a, buf_b, gsem_a,
                  gsem_b, wsem_a, wsem_b):
        wid = lax.axis_index("s") * info.num_cores + lax.axis_index("c")
        base = wid * tpw
        bufs = (buf_a, buf_b)
        gsems = (gsem_a, gsem_b)
        wsems = (wsem_a, wsem_b)
        pltpu.sync_copy(idx_hbm.at[pl.ds(base, tpw)], idx_v)
        gp = [None, None]
        wp = [None, None]
        gp[0] = pltpu.async_copy(
            tab_hbm.at[idx_v.at[0]], bufs[0], gsems[0])
        for k in range(tpw):
            b = k % 2
            nb = (k + 1) % 2
            if k + 1 < tpw:
                if wp[nb] is not None:
                    wp[nb].wait()
                gp[nb] = pltpu.async_copy(
                    tab_hbm.at[idx_v.at[k + 1]], bufs[nb], gsems[nb])
            gp[b].wait()
            wp[b] = pltpu.async_copy(bufs[b], out_hbm.at[base + k], wsems[b])
        for cp in wp:
            if cp is not None:
                cp.wait()

    return sc_gather


def _mlp_body(xnum_ref, cat_ref, gamma_ref, beta_ref, w1n_ref, w1c_ref,
              b1_ref, w2_ref, b2_ref, w3_ref, b3_ref, out_ref, *, bb: int):
    i = pl.program_id(0)
    xn = xnum_ref[...]
    mean = jnp.mean(xn, axis=0, keepdims=True)
    var = jnp.mean(jnp.square(xn - mean), axis=0, keepdims=True)
    inv = lax.rsqrt(var + 1e-5)
    xb = xnum_ref[pl.ds(i * bb, bb), :]
    xb = (xb - mean) * (inv * gamma_ref[...]) + beta_ref[...]
    h = jnp.dot(xb, w1n_ref[...], preferred_element_type=jnp.float32)
    h = h + jnp.dot(cat_ref[...], w1c_ref[...],
                    preferred_element_type=jnp.float32)
    h = jnp.maximum(h + b1_ref[...], 0.0)
    h = jnp.maximum(
        jnp.dot(h, w2_ref[...], preferred_element_type=jnp.float32)
        + b2_ref[...], 0.0)
    out_ref[...] = (jnp.dot(h, w3_ref[...], preferred_element_type=jnp.float32)
                    + b3_ref[...])


def kernel(x_num, x_cat, tables, bn_gamma, bn_beta, W1, b1, W2, b2, W3, b3):
    B, NUM = x_num.shape
    F, V, D = tables.shape
    H = W2.shape[0]
    FP = F + 2
    FPD = FP * D
    nct = FPD // _L
    ntr = B // _R
    ni = _R * FP

    offs = (jnp.arange(F, dtype=jnp.int32) * V)[None, :]
    glob = x_cat + offs
    glob_p = jnp.concatenate(
        [glob, jnp.zeros((B, 2), jnp.int32)], axis=1)
    idx_perm = (glob_p.reshape(ntr, _R, nct, FP // nct)
                .transpose(0, 2, 1, 3).reshape(ntr, ni))
    tab2d = tables.reshape(F * V, D)

    out3 = _make_sc_tile_gather(ntr, ni, D)(idx_perm, tab2d)
    catp = (out3.reshape(ntr, nct, _R, FP // nct, D)
            .transpose(0, 2, 1, 3, 4).reshape(B, FPD))

    w1c_pad = jnp.concatenate(
        [W1[NUM:], jnp.zeros((FPD - (W1.shape[0] - NUM), H), jnp.float32)])

    bb = 2048
    grid = (B // bb,)
    out = pl.pallas_call(
        functools.partial(_mlp_body, bb=bb),
        grid=grid,
        in_specs=[
            pl.BlockSpec((B, NUM), lambda i: (0, 0)),
            pl.BlockSpec((bb, FPD), lambda i: (i, 0)),
            pl.BlockSpec((1, NUM), lambda i: (0, 0)),
            pl.BlockSpec((1, NUM), lambda i: (0, 0)),
            pl.BlockSpec((NUM, H), lambda i: (0, 0)),
            pl.BlockSpec((FPD, H), lambda i: (0, 0)),
            pl.BlockSpec((1, H), lambda i: (0, 0)),
            pl.BlockSpec((H, H), lambda i: (0, 0)),
            pl.BlockSpec((1, H), lambda i: (0, 0)),
            pl.BlockSpec((H, 1), lambda i: (0, 0)),
            pl.BlockSpec((1, 1), lambda i: (0, 0)),
        ],
        out_specs=pl.BlockSpec((bb, 1), lambda i: (i, 0)),
        out_shape=jax.ShapeDtypeStruct((B, 1), jnp.float32),
        compiler_params=pltpu.CompilerParams(
            dimension_semantics=("arbitrary",)),
    )(x_num, catp, bn_gamma.reshape(1, NUM), bn_beta.reshape(1, NUM),
      W1[:NUM], w1c_pad, b1.reshape(1, H), W2, b2.reshape(1, H),
      W3, b3.reshape(1, 1))
    return out


# speedup vs baseline: 1.1042x; 1.1042x over previous
import functools

import jax
import jax.numpy as jnp
from jax import lax
from jax.experimental import pallas as pl
from jax.experimental.pallas import tpu as pltpu
from jax.experimental.pallas import tpu_sc as plsc

_C = 128
_K = 8


def _make_sc_row_gather(n_chunks: int, d: int):
    info = plsc.get_sparse_core_info()
    nw = info.num_cores * info.num_subcores
    assert n_chunks % nw == 0
    cpw = n_chunks // nw
    assert cpw % _K == 0
    ng = cpw // _K

    mesh = plsc.VectorSubcoreMesh(core_axis_name="c", subcore_axis_name="s")

    @functools.partial(
        pl.kernel,
        mesh=mesh,
        compiler_params=pltpu.CompilerParams(use_tc_tiling_on_sc=False),
        out_type=jax.ShapeDtypeStruct((n_chunks, _C, d), jnp.float32),
        scratch_types=[
            pltpu.VMEM((cpw, _C), jnp.int32),
            pltpu.VMEM((_K, _C, d), jnp.float32),
            pltpu.VMEM((_K, _C, d), jnp.float32),
            pltpu.SemaphoreType.DMA,
            pltpu.SemaphoreType.DMA,
            pltpu.SemaphoreType.DMA,
            pltpu.SemaphoreType.DMA,
        ],
    )
    def sc_gather(idx_hbm, tab_hbm, out_hbm, idx_v, buf_a, buf_b, gsem_a,
                  gsem_b, wsem_a, wsem_b):
        wid = lax.axis_index("s") * info.num_cores + lax.axis_index("c")
        base = wid * cpw
        bufs = (buf_a, buf_b)
        gsems = (gsem_a, gsem_b)
        wsems = (wsem_a, wsem_b)
        pltpu.sync_copy(idx_hbm.at[pl.ds(base, cpw)], idx_v)
        wp = [None, None]

        def fire(g, b):
            def body(m, carry):
                pltpu.async_copy(
                    tab_hbm.at[idx_v.at[g * _K + m]], bufs[b].at[m], gsems[b])
                return carry
            lax.fori_loop(0, _K, body, 0)

        def drain(g, b):
            def body(m, carry):
                pltpu.make_async_copy(
                    tab_hbm.at[idx_v.at[g * _K + m]], bufs[b].at[m],
                    gsems[b]).wait()
                return carry
            lax.fori_loop(0, _K, body, 0)

        fire(0, 0)
        for g in range(ng):
            b = g % 2
            nb = (g + 1) % 2
            if g + 1 < ng:
                if wp[nb] is not None:
                    wp[nb].wait()
                fire(g + 1, nb)
            drain(g, b)
            wp[b] = pltpu.async_copy(
                bufs[b], out_hbm.at[pl.ds(base + g * _K, _K)], wsems[b])
        for cp in wp:
            if cp is not None:
                cp.wait()

    return sc_gather


def _mlp_body(*refs, bb: int, nct: int):
    (xnum_ref, gamma_ref, beta_ref, w1n_ref, b1_ref, w2_ref, b2_ref,
     w3_ref, b3_ref) = refs[:9]
    cat_refs = refs[9:9 + nct]
    w1c_refs = refs[9 + nct:9 + 2 * nct]
    out_ref = refs[9 + 2 * nct]
    i = pl.program_id(0)
    xn = xnum_ref[...]
    mean = jnp.mean(xn, axis=0, keepdims=True)
    var = jnp.mean(jnp.square(xn - mean), axis=0, keepdims=True)
    inv = lax.rsqrt(var + 1e-5)
    xb = xnum_ref[pl.ds(i * bb, bb), :]
    xb = (xb - mean) * (inv * gamma_ref[...]) + beta_ref[...]
    h = jnp.dot(xb, w1n_ref[...], preferred_element_type=jnp.float32)
    for cr, wr in zip(cat_refs, w1c_refs):
        h = h + jnp.dot(cr[...], wr[...], preferred_element_type=jnp.float32)
    h = jnp.maximum(h + b1_ref[...], 0.0)
    h = jnp.maximum(
        jnp.dot(h, w2_ref[...], preferred_element_type=jnp.float32)
        + b2_ref[...], 0.0)
    out_ref[...] = (jnp.dot(h, w3_ref[...], preferred_element_type=jnp.float32)
                    + b3_ref[...])


def kernel(x_num, x_cat, tables, bn_gamma, bn_beta, W1, b1, W2, b2, W3, b3):
    B, NUM = x_num.shape
    F, V, D = tables.shape
    H = W2.shape[0]
    FP = F + 2
    FPD = FP * D
    nct = FPD // 128
    qq = 128 // D

    offs = (jnp.arange(F, dtype=jnp.int32) * V)[None, :]
    glob = x_cat + offs
    glob_p = jnp.concatenate(
        [glob, jnp.zeros((B, FP - F), jnp.int32)], axis=1)
    idx = (glob_p.reshape(B, nct, qq).transpose(1, 0, 2)
           .reshape(nct * B * qq // _C, _C))
    tab2d = tables.reshape(F * V, D)

    out3 = _make_sc_row_gather(idx.shape[0], D)(idx, tab2d)
    cats = out3.reshape(nct, B, 128)
    w1c_pad = jnp.concatenate(
        [W1[NUM:], jnp.zeros((FPD - (W1.shape[0] - NUM), H), jnp.float32)])

    bb = 2048
    grid = (B // bb,)
    full = lambda i: (0, 0)
    out = pl.pallas_call(
        functools.partial(_mlp_body, bb=bb, nct=nct),
        grid=grid,
        in_specs=[
            pl.BlockSpec((B, NUM), full),
            pl.BlockSpec((1, NUM), full),
            pl.BlockSpec((1, NUM), full),
            pl.BlockSpec((NUM, H), full),
            pl.BlockSpec((1, H), full),
            pl.BlockSpec((H, H), full),
            pl.BlockSpec((1, H), full),
            pl.BlockSpec((H, 1), full),
            pl.BlockSpec((1, 1), full),
        ] + [pl.BlockSpec((bb, 128), lambda i: (i, 0))] * nct
          + [pl.BlockSpec((128, H), full)] * nct,
        out_specs=pl.BlockSpec((bb, 1), lambda i: (i, 0)),
        out_shape=jax.ShapeDtypeStruct((B, 1), jnp.float32),
        compiler_params=pltpu.CompilerParams(
            dimension_semantics=("arbitrary",)),
    )(x_num, bn_gamma.reshape(1, NUM), bn_beta.reshape(1, NUM), W1[:NUM],
      b1.reshape(1, H), W2, b2.reshape(1, H), W3, b3.reshape(1, 1),
      *[cats[c] for c in range(nct)],
      *[w1c_pad[c * 128:(c + 1) * 128] for c in range(nct)])
    return out


# speedup vs baseline: 1.9273x; 1.7454x over previous
import functools

import jax
import jax.numpy as jnp
from jax import lax
from jax.experimental import pallas as pl
from jax.experimental.pallas import tpu as pltpu
from jax.experimental.pallas import tpu_sc as plsc

_C = 2048
_Q = 4
_L = 128


def _make_sc_plane_gather(f: int, d: int, v: int, b: int):
    info = plsc.get_sparse_core_info()
    nw = info.num_cores * info.num_subcores
    n_planes = f * d
    assert n_planes % nw == 0 and b % _C == 0
    ppw = n_planes // nw
    nc = b // _C

    mesh = plsc.VectorSubcoreMesh(core_axis_name="c", subcore_axis_name="s")

    @functools.partial(
        pl.kernel,
        mesh=mesh,
        compiler_params=pltpu.CompilerParams(use_tc_tiling_on_sc=False),
        out_type=jax.ShapeDtypeStruct((n_planes, b), jnp.float32),
        scratch_types=[
            pltpu.VMEM((nc, _C), jnp.int32),
            pltpu.VMEM((b,), jnp.float32),
            pltpu.VMEM((b,), jnp.float32),
            pltpu.SemaphoreType.DMA,
            pltpu.SemaphoreType.DMA,
            pltpu.SemaphoreType.DMA,
        ],
    )
    def sc_gather(idx_hbm, tab_hbm, out_hbm, idx_v, out_a, out_b, gsem,
                  wsem_a, wsem_b):
        wid = lax.axis_index("s") * info.num_cores + lax.axis_index("c")
        base = wid * ppw
        bufs = (out_a, out_b)
        wsems = (wsem_a, wsem_b)
        pend = [None, None]
        for j in range(ppw):
            p = base + j
            row = tab_hbm.at[p]
            if j == 0:
                pltpu.sync_copy(idx_hbm.at[p // d], idx_v)
            else:
                @pl.when(p % d == 0)
                def _():
                    pltpu.sync_copy(idx_hbm.at[p // d], idx_v)
            buf = bufs[j % 2]
            if pend[j % 2] is not None:
                pend[j % 2].wait()

            def fire_drain(c, carry, row=row, buf=buf):
                pltpu.async_copy(
                    row.at[idx_v.at[c]], buf.at[pl.ds(c * _C, _C)], gsem)

                @pl.when(c >= _Q)
                def _():
                    pltpu.make_async_copy(
                        row.at[idx_v.at[c - _Q]],
                        buf.at[pl.ds((c - _Q) * _C, _C)], gsem).wait()
                return carry

            lax.fori_loop(0, nc, fire_drain, 0)

            def drain(c, carry, row=row, buf=buf):
                pltpu.make_async_copy(
                    row.at[idx_v.at[c]], buf.at[pl.ds(c * _C, _C)],
                    gsem).wait()
                return carry

            lax.fori_loop(nc - _Q, nc, drain, 0)
            pend[j % 2] = pltpu.async_copy(buf, out_hbm.at[p], wsems[j % 2])
        for cp in pend:
            if cp is not None:
                cp.wait()

    return sc_gather


def _mlp_t_body(xn_ref, cat_ref, gamma_ref, beta_ref, w1n_ref, w1c_ref,
                b1_ref, w2_ref, b2_ref, w3_ref, b3_ref, out_ref, *, bb: int):
    i = pl.program_id(0)
    xn = xn_ref[...]
    mean = jnp.mean(xn, axis=1, keepdims=True)
    var = jnp.mean(jnp.square(xn - mean), axis=1, keepdims=True)
    inv = lax.rsqrt(var + 1e-5)
    xb = xn_ref[:, pl.ds(i * bb, bb)]
    xb = (xb - mean) * (inv * gamma_ref[...]) + beta_ref[...]
    h = jnp.dot(w1n_ref[...], xb, preferred_element_type=jnp.float32)
    w1c = w1c_ref[...]
    hc = [jnp.dot(w1c, cat_ref[:, c, :], preferred_element_type=jnp.float32)
          for c in range(bb // _L)]
    h = h + jnp.concatenate(hc, axis=1)
    h = jnp.maximum(h + b1_ref[...], 0.0)
    h = jnp.maximum(
        jnp.dot(w2_ref[...], h, preferred_element_type=jnp.float32)
        + b2_ref[...], 0.0)
    out_ref[...] = (jnp.dot(w3_ref[...], h, preferred_element_type=jnp.float32)
                    + b3_ref[...])


def kernel(x_num, x_cat, tables, bn_gamma, bn_beta, W1, b1, W2, b2, W3, b3):
    B, NUM = x_num.shape
    F, V, D = tables.shape
    H = W2.shape[0]
    FD = F * D

    xnT = x_num.T
    idxT = x_cat.T.reshape(F, B // _C, _C)
    planes = tables.transpose(0, 2, 1).reshape(FD, V)

    catT = _make_sc_plane_gather(F, D, V, B)(idxT, planes)
    cat3 = catT.reshape(FD, B // _L, _L)

    bb = 2048
    grid = (B // bb,)
    outT = pl.pallas_call(
        functools.partial(_mlp_t_body, bb=bb),
        grid=grid,
        in_specs=[
            pl.BlockSpec((NUM, B), lambda i: (0, 0)),
            pl.BlockSpec((FD, bb // _L, _L), lambda i: (0, i, 0)),
            pl.BlockSpec((NUM, 1), lambda i: (0, 0)),
            pl.BlockSpec((NUM, 1), lambda i: (0, 0)),
            pl.BlockSpec((H, NUM), lambda i: (0, 0)),
            pl.BlockSpec((H, FD), lambda i: (0, 0)),
            pl.BlockSpec((H, 1), lambda i: (0, 0)),
            pl.BlockSpec((H, H), lambda i: (0, 0)),
            pl.BlockSpec((H, 1), lambda i: (0, 0)),
            pl.BlockSpec((1, H), lambda i: (0, 0)),
            pl.BlockSpec((1, 1), lambda i: (0, 0)),
        ],
        out_specs=pl.BlockSpec((1, bb), lambda i: (0, i)),
        out_shape=jax.ShapeDtypeStruct((1, B), jnp.float32),
        compiler_params=pltpu.CompilerParams(
            dimension_semantics=("arbitrary",)),
    )(xnT, cat3, bn_gamma.reshape(NUM, 1), bn_beta.reshape(NUM, 1),
      W1[:NUM].T, W1[NUM:].T, b1.reshape(H, 1), W2.T, b2.reshape(H, 1),
      W3.T, b3.reshape(1, 1))
    return outT.reshape(B, 1)
